# Initial kernel scaffold; baseline (speedup 1.0000x reference)
import jax
import jax.numpy as jnp
from jax.experimental import pallas as pl


def kernel(input_ids, token_table, pos_table, tok_ln_w, tok_ln_b, pos_ln_w, pos_ln_b):
    raise NotImplementedError("write your pallas kernel here")


# speedup vs baseline: 1.2365x; 1.2365x over previous
import functools

import jax
import jax.numpy as jnp
from jax import lax
from jax.experimental import pallas as pl
from jax.experimental.pallas import tpu as pltpu
from jax.experimental.pallas import tpu_sc as plsc


def _sc_gather(ids_flat, table):
    N = ids_flat.shape[0]
    V, D = table.shape
    info = plsc.get_sparse_core_info()
    nw = info.num_cores * info.num_subcores
    rows_per_w = N // nw
    ch = 32
    n_ch = rows_per_w // ch
    mesh = plsc.VectorSubcoreMesh(core_axis_name="c", subcore_axis_name="s")

    @functools.partial(
        pl.kernel,
        mesh=mesh,
        out_type=jax.ShapeDtypeStruct((N, D), jnp.float32),
        scratch_types=[
            pltpu.VMEM((ch,), jnp.int32),
            pltpu.VMEM((ch, D), jnp.float32),
            pltpu.SemaphoreType.DMA,
        ],
    )
    def gather_kernel(ids_hbm, table_hbm, out_hbm, idx_v, rows_v, sem):
        wid = lax.axis_index("s") * info.num_cores + lax.axis_index("c")
        base = wid * rows_per_w

        def body(i, carry):
            off = base + i * ch
            pltpu.sync_copy(ids_hbm.at[pl.ds(off, ch)], idx_v)
            pltpu.async_copy(table_hbm.at[idx_v], rows_v, sem).wait()
            pltpu.sync_copy(rows_v, out_hbm.at[pl.ds(off, ch)])
            return carry

        lax.fori_loop(0, n_ch, body, 0)

    return gather_kernel(ids_flat, table)


def _tc_ln_add(gathered, pos_table, tok_w, tok_b, pos_w, pos_b, S):
    N, D = gathered.shape
    blk = 512
    s_blocks = S // blk

    def body(g_ref, p_ref, tw_ref, tb_ref, pw_ref, pb_ref, o_ref):
        x = g_ref[...]
        mu = jnp.mean(x, axis=-1, keepdims=True)
        var = jnp.mean((x - mu) ** 2, axis=-1, keepdims=True)
        tok = (x - mu) * lax.rsqrt(var + 1e-5) * tw_ref[...] + tb_ref[...]
        p = p_ref[...]
        pmu = jnp.mean(p, axis=-1, keepdims=True)
        pvar = jnp.mean((p - pmu) ** 2, axis=-1, keepdims=True)
        pos = (p - pmu) * lax.rsqrt(pvar + 1e-5) * pw_ref[...] + pb_ref[...]
        o_ref[...] = tok + pos

    vec = lambda: pl.BlockSpec((1, D), lambda i: (0, 0))
    return pl.pallas_call(
        body,
        grid=(N // blk,),
        in_specs=[
            pl.BlockSpec((blk, D), lambda i: (i, 0)),
            pl.BlockSpec((blk, D), lambda i: (i % s_blocks, 0)),
            vec(), vec(), vec(), vec(),
        ],
        out_specs=pl.BlockSpec((blk, D), lambda i: (i, 0)),
        out_shape=jax.ShapeDtypeStruct((N, D), jnp.float32),
    )(gathered, pos_table, tok_w.reshape(1, D), tok_b.reshape(1, D),
      pos_w.reshape(1, D), pos_b.reshape(1, D))


def kernel(input_ids, token_table, pos_table, tok_ln_w, tok_ln_b, pos_ln_w, pos_ln_b):
    B, S = input_ids.shape
    V, D = token_table.shape
    ids_flat = input_ids.reshape(B * S).astype(jnp.int32)
    gathered = _sc_gather(ids_flat, token_table)
    out = _tc_ln_add(gathered, pos_table, tok_ln_w, tok_ln_b, pos_ln_w, pos_ln_b, S)
    return out.reshape(B, S, D)


# speedup vs baseline: 1.2777x; 1.0333x over previous
import functools

import jax
import jax.numpy as jnp
from jax import lax
from jax.experimental import pallas as pl
from jax.experimental.pallas import tpu as pltpu
from jax.experimental.pallas import tpu_sc as plsc


def _sc_gather(ids_flat, table):
    N = ids_flat.shape[0]
    V, D = table.shape
    info = plsc.get_sparse_core_info()
    nw = info.num_cores * info.num_subcores
    rows_per_w = N // nw
    ch = 32
    n_ch = rows_per_w // ch
    mesh = plsc.VectorSubcoreMesh(core_axis_name="c", subcore_axis_name="s")

    @functools.partial(
        pl.kernel,
        mesh=mesh,
        out_type=jax.ShapeDtypeStruct((N, D), jnp.float32),
        scratch_types=[
            pltpu.VMEM((ch,), jnp.int32),
            pltpu.VMEM((ch, D), jnp.float32),
            pltpu.SemaphoreType.DMA,
        ],
    )
    def gather_kernel(ids_hbm, table_hbm, out_hbm, idx_v, rows_v, sem):
        wid = lax.axis_index("s") * info.num_cores + lax.axis_index("c")
        base = wid * rows_per_w

        def body(i, carry):
            off = base + i * ch
            pltpu.sync_copy(ids_hbm.at[pl.ds(off, ch)], idx_v)
            pltpu.async_copy(table_hbm.at[idx_v], rows_v, sem).wait()
            pltpu.sync_copy(rows_v, out_hbm.at[pl.ds(off, ch)])
            return carry

        lax.fori_loop(0, n_ch, body, 0)

    return gather_kernel(ids_flat, table)


def _tc_ln_add(gathered, pos_table, tok_w, tok_b, pos_w, pos_b, S):
    N, D = gathered.shape
    blk = 512
    s_blocks = S // blk

    def body(g_ref, p_ref, tw_ref, tb_ref, pw_ref, pb_ref, o_ref):
        x = g_ref[...]
        mu = jnp.mean(x, axis=-1, keepdims=True)
        var = jnp.mean((x - mu) ** 2, axis=-1, keepdims=True)
        tok = (x - mu) * lax.rsqrt(var + 1e-5) * tw_ref[...] + tb_ref[...]
        p = p_ref[...]
        pmu = jnp.mean(p, axis=-1, keepdims=True)
        pvar = jnp.mean((p - pmu) ** 2, axis=-1, keepdims=True)
        pos = (p - pmu) * lax.rsqrt(pvar + 1e-5) * pw_ref[...] + pb_ref[...]
        o_ref[...] = tok + pos

    nb = N // blk // s_blocks
    vec = lambda: pl.BlockSpec((1, D), lambda s, b: (0, 0))
    return pl.pallas_call(
        body,
        grid=(s_blocks, nb),
        in_specs=[
            pl.BlockSpec((blk, D), lambda s, b: (b * s_blocks + s, 0)),
            pl.BlockSpec((blk, D), lambda s, b: (s, 0)),
            vec(), vec(), vec(), vec(),
        ],
        out_specs=pl.BlockSpec((blk, D), lambda s, b: (b * s_blocks + s, 0)),
        out_shape=jax.ShapeDtypeStruct((N, D), jnp.float32),
    )(gathered, pos_table, tok_w.reshape(1, D), tok_b.reshape(1, D),
      pos_w.reshape(1, D), pos_b.reshape(1, D))


def kernel(input_ids, token_table, pos_table, tok_ln_w, tok_ln_b, pos_ln_w, pos_ln_b):
    B, S = input_ids.shape
    V, D = token_table.shape
    ids_flat = input_ids.reshape(B * S).astype(jnp.int32)
    gathered = _sc_gather(ids_flat, token_table)
    out = _tc_ln_add(gathered, pos_table, tok_ln_w, tok_ln_b, pos_ln_w, pos_ln_b, S)
    return out.reshape(B, S, D)
